# baseline (speedup 1.0000x reference)
import functools

import jax
import jax.numpy as jnp
from jax import lax
from jax.experimental import pallas as pl
from jax.experimental.pallas import tpu as pltpu
from jax.experimental.pallas import tpu_sc as plsc

N_NODES = 100000
D_FEAT = 128
BATCH = 50000
LANES = 16

NC, NS = 2, 16
NW = NC * NS
NBUF = 4


def _mean_agg_kernel(nchunks, nb, s, last_base, features_hbm, idx_hbm,
                     out_hbm, idx_v, rows_v, out_v, gsems, osems):
    wid = lax.axis_index("s") * NC + lax.axis_index("c")
    rows_per_worker = nchunks * nb
    base = jnp.minimum(wid * rows_per_worker, last_base)
    pltpu.sync_copy(idx_hbm.at[pl.ds(base * s, rows_per_worker * s)], idx_v)

    inv_s = jnp.float32(1.0 / s)
    iota = lax.iota(jnp.int32, LANES)
    hi_mask = jnp.full((LANES,), -65536, jnp.int32)

    for par in range(NBUF):
        pltpu.async_copy(
            features_hbm.at[idx_v.at[pl.ds(par * nb * s, nb * s)]],
            rows_v.at[par], gsems[par])

    def body(i, carry):
        cb = i * NBUF
        for par in range(NBUF):
            c = cb + par
            pltpu.make_async_copy(
                features_hbm.at[idx_v.at[pl.ds(c * nb * s, nb * s)]],
                rows_v.at[par], gsems[par]).wait()
            @pl.when(c >= NBUF)
            def _():
                pltpu.make_async_copy(out_v.at[par],
                                      out_hbm.at[pl.ds(0, nb)],
                                      osems[par]).wait()
            out_p = out_v.at[par]
            for r in range(nb):
                row_idx = iota * 0 + r
                for d in range(D_FEAT // (2 * LANES)):
                    sl = pl.ds(d * LANES, LANES)
                    v = rows_v[par, r * s, sl]
                    ae = plsc.bitcast(v << 16, jnp.float32)
                    ao = plsc.bitcast(v & hi_mask, jnp.float32)
                    for j in range(1, s):
                        v = rows_v[par, r * s + j, sl]
                        ae = ae + plsc.bitcast(v << 16, jnp.float32)
                        ao = ao + plsc.bitcast(v & hi_mask, jnp.float32)
                    cols = iota * 2 + (d * 2 * LANES)
                    plsc.store_scatter(out_p, [row_idx, cols], ae * inv_s)
                    plsc.store_scatter(out_p, [row_idx, cols + 1],
                                       ao * inv_s)
            @pl.when(c + NBUF < nchunks)
            def _():
                pltpu.async_copy(
                    features_hbm.at[
                        idx_v.at[pl.ds((c + NBUF) * nb * s, nb * s)]],
                    rows_v.at[par], gsems[par])
            pltpu.async_copy(out_v.at[par],
                             out_hbm.at[pl.ds(base + c * nb, nb)],
                             osems[par])
        return carry

    lax.fori_loop(0, nchunks // NBUF, body, 0)

    for par in range(NBUF):
        pltpu.make_async_copy(out_v.at[par], out_hbm.at[pl.ds(0, nb)],
                              osems[par]).wait()


def kernel(features, neigh_idx, num_sample):
    del num_sample
    b, s = neigh_idx.shape
    nb = (128 // s) // 8 * 8
    assert nb >= 8
    nchunks = -(-b // (NW * nb))
    nchunks = -(-nchunks // NBUF) * NBUF
    rows_per_worker = nchunks * nb
    last_base = (b - rows_per_worker) // 8 * 8
    assert last_base >= 0 and (NW - 1) * rows_per_worker + rows_per_worker >= b

    idx_flat = neigh_idx.reshape(-1).astype(jnp.int32)
    n = features.shape[0]
    feats_packed = jax.lax.bitcast_convert_type(
        features.astype(jnp.bfloat16).reshape(n, D_FEAT // 2, 2), jnp.int32)
    feats_packed = jnp.concatenate(
        [feats_packed, jnp.zeros((n, D_FEAT // 2), jnp.int32)], axis=1)

    mesh = plsc.VectorSubcoreMesh(core_axis_name="c", subcore_axis_name="s",
                                  num_cores=NC, num_subcores=NS)
    out = pl.kernel(
        functools.partial(_mean_agg_kernel, nchunks, nb, s, last_base),
        out_type=jax.ShapeDtypeStruct((b, D_FEAT), jnp.float32),
        mesh=mesh,
        compiler_params=pltpu.CompilerParams(needs_layout_passes=False),
        scratch_types=[
            pltpu.VMEM((rows_per_worker * s,), jnp.int32),
            pltpu.VMEM((NBUF, nb * s, D_FEAT), jnp.int32),
            pltpu.VMEM((NBUF, nb, D_FEAT), jnp.float32),
            [pltpu.SemaphoreType.DMA] * NBUF,
            [pltpu.SemaphoreType.DMA] * NBUF,
        ],
    )(feats_packed, idx_flat)
    return out


# speedup vs baseline: 2.6734x; 2.6734x over previous
import functools

import jax
import jax.numpy as jnp
from jax import lax
from jax.experimental import pallas as pl
from jax.experimental.pallas import tpu as pltpu
from jax.experimental.pallas import tpu_sc as plsc

N_NODES = 100000
D_FEAT = 128
BATCH = 50000
LANES = 16

NC, NS = 2, 16
NW = NC * NS
NBUF = 4


def _mean_agg_kernel(nchunks, nb, s, last_base, features_hbm, idx_hbm,
                     out_hbm, idx_v, rows_v, out_v, gsems, osems):
    wid = lax.axis_index("s") * NC + lax.axis_index("c")
    rows_per_worker = nchunks * nb
    base = jnp.minimum(wid * rows_per_worker, last_base)
    pltpu.sync_copy(idx_hbm.at[pl.ds(base * s, rows_per_worker * s)], idx_v)

    inv_s = jnp.float32(1.0 / s)
    hi_mask = jnp.full((LANES,), -65536, jnp.int32)

    for par in range(NBUF):
        pltpu.async_copy(
            features_hbm.at[idx_v.at[pl.ds(par * nb * s, nb * s)]],
            rows_v.at[par], gsems[par])

    def body(i, carry):
        cb = i * NBUF
        for par in range(NBUF):
            c = cb + par
            pltpu.make_async_copy(
                features_hbm.at[idx_v.at[pl.ds(c * nb * s, nb * s)]],
                rows_v.at[par], gsems[par]).wait()
            @pl.when(c >= NBUF)
            def _():
                pltpu.make_async_copy(out_v.at[par],
                                      out_hbm.at[pl.ds(0, nb)],
                                      osems[par]).wait()
            for r in range(nb):
                for d in range(D_FEAT // (2 * LANES)):
                    sl = pl.ds(d * LANES, LANES)
                    v = rows_v[par, r * s, sl]
                    lo = plsc.bitcast(v << 16, jnp.float32)
                    hi = plsc.bitcast(v & hi_mask, jnp.float32)
                    for j in range(1, s):
                        v = rows_v[par, r * s + j, sl]
                        lo = lo + plsc.bitcast(v << 16, jnp.float32)
                        hi = hi + plsc.bitcast(v & hi_mask, jnp.float32)
                    out_v[par, r, pl.ds(d * LANES, LANES)] = lo * inv_s
                    out_v[par, r, pl.ds(D_FEAT // 2 + d * LANES, LANES)] = (
                        hi * inv_s)
            @pl.when(c + NBUF < nchunks)
            def _():
                pltpu.async_copy(
                    features_hbm.at[
                        idx_v.at[pl.ds((c + NBUF) * nb * s, nb * s)]],
                    rows_v.at[par], gsems[par])
            pltpu.async_copy(out_v.at[par],
                             out_hbm.at[pl.ds(base + c * nb, nb)],
                             osems[par])
        return carry

    lax.fori_loop(0, nchunks // NBUF, body, 0)

    for par in range(NBUF):
        pltpu.make_async_copy(out_v.at[par], out_hbm.at[pl.ds(0, nb)],
                              osems[par]).wait()


def kernel(features, neigh_idx, num_sample):
    del num_sample
    b, s = neigh_idx.shape
    nb = (128 // s) // 8 * 8
    assert nb >= 8
    nchunks = -(-b // (NW * nb))
    nchunks = -(-nchunks // NBUF) * NBUF
    rows_per_worker = nchunks * nb
    last_base = (b - rows_per_worker) // 8 * 8
    assert last_base >= 0 and (NW - 1) * rows_per_worker + rows_per_worker >= b

    idx_flat = neigh_idx.reshape(-1).astype(jnp.int32)
    bits = jax.lax.bitcast_convert_type(features, jnp.uint32)

    def rnd16(t):
        return (t + jnp.uint32(0x7FFF) + ((t >> 16) & jnp.uint32(1))) >> 16

    half = D_FEAT // 2
    packed = rnd16(bits[:, :half]) | (rnd16(bits[:, half:]) << 16)
    feats_packed = jnp.concatenate(
        [jax.lax.bitcast_convert_type(packed, jnp.int32),
         jnp.zeros((features.shape[0], half), jnp.int32)], axis=1)

    mesh = plsc.VectorSubcoreMesh(core_axis_name="c", subcore_axis_name="s",
                                  num_cores=NC, num_subcores=NS)
    out = pl.kernel(
        functools.partial(_mean_agg_kernel, nchunks, nb, s, last_base),
        out_type=jax.ShapeDtypeStruct((b, D_FEAT), jnp.float32),
        mesh=mesh,
        compiler_params=pltpu.CompilerParams(needs_layout_passes=False),
        scratch_types=[
            pltpu.VMEM((rows_per_worker * s,), jnp.int32),
            pltpu.VMEM((NBUF, nb * s, D_FEAT), jnp.int32),
            pltpu.VMEM((NBUF, nb, D_FEAT), jnp.float32),
            [pltpu.SemaphoreType.DMA] * NBUF,
            [pltpu.SemaphoreType.DMA] * NBUF,
        ],
    )(feats_packed, idx_flat)
    return out
